# baseline (device time: 119564 ns/iter reference)
import jax
import jax.numpy as jnp
from jax import lax
from jax.experimental import pallas as pl
from jax.experimental.pallas import tpu as pltpu

N_DEV = 4
S_LOC = 1024
H = 8
D = 128
BLK = 64
SCALE = 0.08838834764831843
BF = jnp.bfloat16


def kernel(x, Wq, K_ext, V_ext, Wo):
    def body(x_ref, wq_ref, k_ref, v_ref, wo_ref, out_ref,
             kv_scr, stage, q_scr, acc_scr, wsum_scr,
             copy_sem, send_r, recv_r, send_l, recv_l):
        my = lax.axis_index("i")
        left = (my - 1) % N_DEV
        right = (my + 1) % N_DEV

        cp = pltpu.make_async_copy(k_ref, stage, copy_sem)
        cp.start()
        cp.wait()
        kv_scr[0, 0] = jnp.reshape(stage[0], (S_LOC, H * D)).astype(BF)
        cp = pltpu.make_async_copy(v_ref, stage, copy_sem)
        cp.start()
        cp.wait()
        kv_scr[0, 1] = jnp.reshape(stage[0], (S_LOC, H * D)).astype(BF)

        barrier_sem = pltpu.get_barrier_semaphore()
        for nbr in (left, right):
            pl.semaphore_signal(barrier_sem, inc=1, device_id=(nbr,),
                                device_id_type=pl.DeviceIdType.MESH)
        pl.semaphore_wait(barrier_sem, 2)

        r0 = pltpu.make_async_remote_copy(
            src_ref=kv_scr.at[0], dst_ref=kv_scr.at[1],
            send_sem=send_r.at[0], recv_sem=recv_r.at[0],
            device_id=(right,), device_id_type=pl.DeviceIdType.MESH)
        r0.start()
        l0 = pltpu.make_async_remote_copy(
            src_ref=kv_scr.at[0], dst_ref=kv_scr.at[3],
            send_sem=send_l.at[0], recv_sem=recv_l.at[0],
            device_id=(left,), device_id_type=pl.DeviceIdType.MESH)
        l0.start()

        q_scr[...] = (jnp.dot(x_ref[0].astype(BF), wq_ref[...].astype(BF),
                              preferred_element_type=jnp.float32)
                      * SCALE).astype(BF)

        acc_scr[...] = jnp.zeros((S_LOC, H * D), jnp.float32)
        wsum_scr[...] = jnp.zeros((S_LOC, H), jnp.float32)

        R = S_LOC // 2
        ones_col = jnp.ones((S_LOC, 1), BF)

        def fold_chunk(slot):
            origin = (my - slot) % N_DEV
            kb = (origin * S_LOC
                  + lax.broadcasted_iota(jnp.int32, (1, S_LOC), 1)) // BLK
            for r in range(S_LOC // R):
                rs = slice(r * R, (r + 1) * R)
                qb = (my * S_LOC + r * R
                      + lax.broadcasted_iota(jnp.int32, (R, 1), 0)) // BLK
                mask = (qb == kb) | (kb == 0) | ((qb + kb) % 3 == 0)
                bias = jnp.where(mask, 0.0, -1e9)
                for hd in range(H):
                    sl = slice(hd * D, (hd + 1) * D)
                    s = lax.dot_general(q_scr[rs, sl], kv_scr[slot, 0, :, sl],
                                        (((1,), (1,)), ((), ())),
                                        preferred_element_type=jnp.float32)
                    w = jnp.exp((s + bias).astype(BF))
                    wsum_scr[rs, hd:hd + 1] = (
                        wsum_scr[rs, hd:hd + 1]
                        + lax.dot_general(w, ones_col,
                                          (((1,), (0,)), ((), ())),
                                          preferred_element_type=jnp.float32))
                    acc_scr[rs, sl] = acc_scr[rs, sl] + lax.dot_general(
                        w, kv_scr[slot, 1, :, sl],
                        (((1,), (0,)), ((), ())),
                        preferred_element_type=jnp.float32)

        fold_chunk(0)

        r0.wait()
        l0.wait()

        r1 = pltpu.make_async_remote_copy(
            src_ref=kv_scr.at[1, 0], dst_ref=kv_scr.at[2, 0],
            send_sem=send_r.at[1], recv_sem=recv_r.at[1],
            device_id=(right,), device_id_type=pl.DeviceIdType.MESH)
        r1.start()
        l1 = pltpu.make_async_remote_copy(
            src_ref=kv_scr.at[3, 1], dst_ref=kv_scr.at[2, 1],
            send_sem=send_l.at[1], recv_sem=recv_l.at[1],
            device_id=(left,), device_id_type=pl.DeviceIdType.MESH)
        l1.start()

        fold_chunk(1)
        fold_chunk(3)

        r1.wait()
        l1.wait()

        fold_chunk(2)

        out = jnp.zeros((S_LOC, H * D), jnp.float32)
        for hd in range(H):
            sl = slice(hd * D, (hd + 1) * D)
            ctx = (acc_scr[:, sl] / wsum_scr[:, hd:hd + 1]).astype(BF)
            out = out + jnp.dot(ctx, wo_ref[sl, :].astype(BF),
                                preferred_element_type=jnp.float32)
        out_ref[0] = out

    return pl.pallas_call(
        body,
        out_shape=jax.ShapeDtypeStruct((1, S_LOC, H * D), jnp.float32),
        in_specs=[
            pl.BlockSpec(memory_space=pltpu.VMEM),
            pl.BlockSpec(memory_space=pltpu.VMEM),
            pl.BlockSpec(memory_space=pl.ANY),
            pl.BlockSpec(memory_space=pl.ANY),
            pl.BlockSpec(memory_space=pltpu.VMEM),
        ],
        out_specs=pl.BlockSpec(memory_space=pltpu.VMEM),
        scratch_shapes=[
            pltpu.VMEM((N_DEV, 2, S_LOC, H * D), BF),
            pltpu.VMEM((1, S_LOC, H, D), jnp.float32),
            pltpu.VMEM((S_LOC, H * D), BF),
            pltpu.VMEM((S_LOC, H * D), jnp.float32),
            pltpu.VMEM((S_LOC, H), jnp.float32),
            pltpu.SemaphoreType.DMA,
            pltpu.SemaphoreType.DMA((2,)),
            pltpu.SemaphoreType.DMA((2,)),
            pltpu.SemaphoreType.DMA((2,)),
            pltpu.SemaphoreType.DMA((2,)),
        ],
        compiler_params=pltpu.CompilerParams(
            collective_id=0,
            vmem_limit_bytes=100 * 1024 * 1024,
        ),
    )(x, Wq, K_ext, V_ext, Wo)


# device time: 114931 ns/iter; 1.0403x vs baseline; 1.0403x over previous
import jax
import jax.numpy as jnp
from jax import lax
from jax.experimental import pallas as pl
from jax.experimental.pallas import tpu as pltpu

N_DEV = 4
S_LOC = 1024
H = 8
D = 128
BLK = 64
SCALE = 0.08838834764831843
BF = jnp.bfloat16


def kernel(x, Wq, K_ext, V_ext, Wo):
    def body(x_ref, wq_ref, k_ref, v_ref, wo_ref, out_ref,
             kv_scr, stage, q_scr, acc_scr, wsum_scr,
             copy_sem, send_r, recv_r, send_l, recv_l):
        my = lax.axis_index("i")
        left = (my - 1) % N_DEV
        right = (my + 1) % N_DEV

        cp = pltpu.make_async_copy(k_ref, stage, copy_sem)
        cp.start()
        cp.wait()
        kv_scr[0, 0] = jnp.reshape(stage[0], (S_LOC, H * D)).astype(BF)
        cp = pltpu.make_async_copy(v_ref, stage, copy_sem)
        cp.start()
        cp.wait()
        kv_scr[0, 1] = jnp.reshape(stage[0], (S_LOC, H * D)).astype(BF)

        barrier_sem = pltpu.get_barrier_semaphore()
        for nbr in (left, right):
            pl.semaphore_signal(barrier_sem, inc=1, device_id=(nbr,),
                                device_id_type=pl.DeviceIdType.MESH)
        pl.semaphore_wait(barrier_sem, 2)

        r0 = pltpu.make_async_remote_copy(
            src_ref=kv_scr.at[0], dst_ref=kv_scr.at[1],
            send_sem=send_r.at[0], recv_sem=recv_r.at[0],
            device_id=(right,), device_id_type=pl.DeviceIdType.MESH)
        r0.start()
        l0 = pltpu.make_async_remote_copy(
            src_ref=kv_scr.at[0], dst_ref=kv_scr.at[3],
            send_sem=send_l.at[0], recv_sem=recv_l.at[0],
            device_id=(left,), device_id_type=pl.DeviceIdType.MESH)
        l0.start()

        q_scr[...] = (jnp.dot(x_ref[0].astype(BF), wq_ref[...].astype(BF),
                              preferred_element_type=jnp.float32)
                      * SCALE).astype(BF)

        acc_scr[...] = jnp.zeros((S_LOC, H * D), jnp.float32)
        wsum_scr[...] = jnp.zeros((S_LOC, H), jnp.float32)

        R = S_LOC // 2

        def fold_chunk(slot):
            origin = (my - slot) % N_DEV
            kb = (origin * S_LOC
                  + lax.broadcasted_iota(jnp.int32, (1, S_LOC), 1)) // BLK
            for r in range(S_LOC // R):
                rs = slice(r * R, (r + 1) * R)
                qb = (my * S_LOC + r * R
                      + lax.broadcasted_iota(jnp.int32, (R, 1), 0)) // BLK
                mask = (qb == kb) | (kb == 0) | ((qb + kb) % 3 == 0)
                for hd in range(H):
                    sl = slice(hd * D, (hd + 1) * D)
                    s = lax.dot_general(q_scr[rs, sl], kv_scr[slot, 0, :, sl],
                                        (((1,), (1,)), ((), ())),
                                        preferred_element_type=jnp.float32)
                    w = jnp.where(mask, jnp.exp(s), 0.0)
                    wsum_scr[rs, hd:hd + 1] = (
                        wsum_scr[rs, hd:hd + 1]
                        + jnp.sum(w, axis=1, keepdims=True))
                    acc_scr[rs, sl] = acc_scr[rs, sl] + lax.dot_general(
                        w.astype(BF), kv_scr[slot, 1, :, sl],
                        (((1,), (0,)), ((), ())),
                        preferred_element_type=jnp.float32)

        fold_chunk(0)

        r0.wait()
        l0.wait()

        r1 = pltpu.make_async_remote_copy(
            src_ref=kv_scr.at[1, 0], dst_ref=kv_scr.at[2, 0],
            send_sem=send_r.at[1], recv_sem=recv_r.at[1],
            device_id=(right,), device_id_type=pl.DeviceIdType.MESH)
        r1.start()
        l1 = pltpu.make_async_remote_copy(
            src_ref=kv_scr.at[3, 1], dst_ref=kv_scr.at[2, 1],
            send_sem=send_l.at[1], recv_sem=recv_l.at[1],
            device_id=(left,), device_id_type=pl.DeviceIdType.MESH)
        l1.start()

        fold_chunk(1)
        fold_chunk(3)

        r1.wait()
        l1.wait()

        fold_chunk(2)

        out = jnp.zeros((S_LOC, H * D), jnp.float32)
        for hd in range(H):
            sl = slice(hd * D, (hd + 1) * D)
            ctx = (acc_scr[:, sl] / wsum_scr[:, hd:hd + 1]).astype(BF)
            out = out + jnp.dot(ctx, wo_ref[sl, :].astype(BF),
                                preferred_element_type=jnp.float32)
        out_ref[0] = out

    return pl.pallas_call(
        body,
        out_shape=jax.ShapeDtypeStruct((1, S_LOC, H * D), jnp.float32),
        in_specs=[
            pl.BlockSpec(memory_space=pltpu.VMEM),
            pl.BlockSpec(memory_space=pltpu.VMEM),
            pl.BlockSpec(memory_space=pl.ANY),
            pl.BlockSpec(memory_space=pl.ANY),
            pl.BlockSpec(memory_space=pltpu.VMEM),
        ],
        out_specs=pl.BlockSpec(memory_space=pltpu.VMEM),
        scratch_shapes=[
            pltpu.VMEM((N_DEV, 2, S_LOC, H * D), BF),
            pltpu.VMEM((1, S_LOC, H, D), jnp.float32),
            pltpu.VMEM((S_LOC, H * D), BF),
            pltpu.VMEM((S_LOC, H * D), jnp.float32),
            pltpu.VMEM((S_LOC, H), jnp.float32),
            pltpu.SemaphoreType.DMA,
            pltpu.SemaphoreType.DMA((2,)),
            pltpu.SemaphoreType.DMA((2,)),
            pltpu.SemaphoreType.DMA((2,)),
            pltpu.SemaphoreType.DMA((2,)),
        ],
        compiler_params=pltpu.CompilerParams(
            collective_id=0,
            vmem_limit_bytes=100 * 1024 * 1024,
        ),
    )(x, Wq, K_ext, V_ext, Wo)


# device time: 112596 ns/iter; 1.0619x vs baseline; 1.0207x over previous
import jax
import jax.numpy as jnp
from jax import lax
from jax.experimental import pallas as pl
from jax.experimental.pallas import tpu as pltpu

N_DEV = 4
S_LOC = 1024
HF = S_LOC // 2
H = 8
D = 128
BLK = 64
SCALE = 0.08838834764831843
BF = jnp.bfloat16


def kernel(x, Wq, K_ext, V_ext, Wo):
    def body(x_ref, wq_ref, k_ref, v_ref, wo_ref, out_ref,
             kv_scr, stage, q_scr, acc_scr, wsum_scr,
             copy_sem, send_r, recv_r, send_l, recv_l):
        my = lax.axis_index("i")
        left = (my - 1) % N_DEV
        right = (my + 1) % N_DEV

        QR = HF // 2
        for kv_i, ref in ((0, k_ref), (1, v_ref)):
            for part in range(4):
                cp = pltpu.make_async_copy(
                    ref.at[:, pl.ds(part * QR, QR)], stage, copy_sem)
                cp.start()
                cp.wait()
                quarter = 2 * (part // 2) + kv_i
                kv_scr[0, quarter, (part % 2) * QR:(part % 2 + 1) * QR] = (
                    jnp.reshape(stage[0], (QR, H * D)).astype(BF))

        barrier_sem = pltpu.get_barrier_semaphore()
        for nbr in (left, right):
            pl.semaphore_signal(barrier_sem, inc=1, device_id=(nbr,),
                                device_id_type=pl.DeviceIdType.MESH)
        pl.semaphore_wait(barrier_sem, 2)

        def send(src_slot, src_q, dst_slot, sems, idx, dev):
            c = pltpu.make_async_remote_copy(
                src_ref=kv_scr.at[src_slot, src_q:src_q + 2],
                dst_ref=kv_scr.at[dst_slot, src_q:src_q + 2],
                send_sem=sems[0].at[idx], recv_sem=sems[1].at[idx],
                device_id=(dev,), device_id_type=pl.DeviceIdType.MESH)
            c.start()
            return c

        R_SEMS = (send_r, recv_r)
        L_SEMS = (send_l, recv_l)

        r0a = send(0, 0, 1, R_SEMS, 0, right)
        l0b = send(0, 2, 3, L_SEMS, 0, left)
        r0b = send(0, 2, 1, R_SEMS, 1, right)
        l0a = send(0, 0, 3, L_SEMS, 1, left)

        q_scr[...] = (jnp.dot(x_ref[0].astype(BF), wq_ref[...].astype(BF),
                              preferred_element_type=jnp.float32)
                      * SCALE).astype(BF)

        acc_scr[...] = jnp.zeros((S_LOC, H * D), jnp.float32)
        wsum_scr[...] = jnp.zeros((S_LOC, H), jnp.float32)

        R = S_LOC // 2

        def fold(slot, half):
            origin = (my - slot) % N_DEV
            kb = (origin * S_LOC + half * HF
                  + lax.broadcasted_iota(jnp.int32, (1, HF), 1)) // BLK
            for r in range(S_LOC // R):
                rs = slice(r * R, (r + 1) * R)
                qb = (my * S_LOC + r * R
                      + lax.broadcasted_iota(jnp.int32, (R, 1), 0)) // BLK
                mask = (qb == kb) | (kb == 0) | ((qb + kb) % 3 == 0)
                for hd in range(H):
                    sl = slice(hd * D, (hd + 1) * D)
                    s = lax.dot_general(
                        q_scr[rs, sl], kv_scr[slot, 2 * half, :, sl],
                        (((1,), (1,)), ((), ())),
                        preferred_element_type=jnp.float32)
                    w = jnp.where(mask, jnp.exp(s), 0.0)
                    wsum_scr[rs, hd:hd + 1] = (
                        wsum_scr[rs, hd:hd + 1]
                        + jnp.sum(w, axis=1, keepdims=True))
                    acc_scr[rs, sl] = acc_scr[rs, sl] + lax.dot_general(
                        w.astype(BF), kv_scr[slot, 2 * half + 1, :, sl],
                        (((1,), (0,)), ((), ())),
                        preferred_element_type=jnp.float32)

        fold(0, 0)
        fold(0, 1)

        r0a.wait()
        l0b.wait()
        r1 = send(1, 0, 2, R_SEMS, 2, right)
        l1 = send(3, 2, 2, L_SEMS, 2, left)
        fold(1, 0)
        fold(3, 1)

        r0b.wait()
        l0a.wait()
        fold(1, 1)
        fold(3, 0)

        r1.wait()
        l1.wait()
        fold(2, 0)
        fold(2, 1)

        out_ref[0] = jnp.zeros((S_LOC, H * D), jnp.float32)
        for hd in range(H):
            sl = slice(hd * D, (hd + 1) * D)
            ctx = (acc_scr[:, sl] / wsum_scr[:, hd:hd + 1]).astype(BF)
            out_ref[0] = out_ref[0] + jnp.dot(
                ctx, wo_ref[sl, :].astype(BF),
                preferred_element_type=jnp.float32)

    return pl.pallas_call(
        body,
        out_shape=jax.ShapeDtypeStruct((1, S_LOC, H * D), jnp.float32),
        in_specs=[
            pl.BlockSpec(memory_space=pltpu.VMEM),
            pl.BlockSpec(memory_space=pltpu.VMEM),
            pl.BlockSpec(memory_space=pl.ANY),
            pl.BlockSpec(memory_space=pl.ANY),
            pl.BlockSpec(memory_space=pltpu.VMEM),
        ],
        out_specs=pl.BlockSpec(memory_space=pltpu.VMEM),
        scratch_shapes=[
            pltpu.VMEM((N_DEV, 4, HF, H * D), BF),
            pltpu.VMEM((1, HF // 2, H, D), jnp.float32),
            pltpu.VMEM((S_LOC, H * D), BF),
            pltpu.VMEM((S_LOC, H * D), jnp.float32),
            pltpu.VMEM((S_LOC, H), jnp.float32),
            pltpu.SemaphoreType.DMA,
            pltpu.SemaphoreType.DMA((3,)),
            pltpu.SemaphoreType.DMA((3,)),
            pltpu.SemaphoreType.DMA((3,)),
            pltpu.SemaphoreType.DMA((3,)),
        ],
        compiler_params=pltpu.CompilerParams(
            collective_id=0,
            vmem_limit_bytes=100 * 1024 * 1024,
        ),
    )(x, Wq, K_ext, V_ext, Wo)


# device time: 111543 ns/iter; 1.0719x vs baseline; 1.0094x over previous
import jax
import jax.numpy as jnp
from jax import lax
from jax.experimental import pallas as pl
from jax.experimental.pallas import tpu as pltpu

N_DEV = 4
S_LOC = 1024
HF = S_LOC // 2
H = 8
D = 128
BLK = 64
SCALE = 0.08838834764831843
BF = jnp.bfloat16


def kernel(x, Wq, K_ext, V_ext, Wo):
    def body(x_ref, wq_ref, k_ref, v_ref, wo_ref, out_ref,
             kv_scr, stage, q_scr, acc_scr, wsum_scr,
             copy_sem, send_r, recv_r, send_l, recv_l):
        my = lax.axis_index("i")
        left = (my - 1) % N_DEV
        right = (my + 1) % N_DEV

        QR = HF // 2
        for kv_i, ref in ((0, k_ref), (1, v_ref)):
            for part in range(4):
                cp = pltpu.make_async_copy(
                    ref.at[:, pl.ds(part * QR, QR)], stage, copy_sem)
                cp.start()
                cp.wait()
                quarter = 2 * (part // 2) + kv_i
                kv_scr[0, quarter, (part % 2) * QR:(part % 2 + 1) * QR] = (
                    jnp.reshape(stage[0], (QR, H * D)).astype(BF))

        barrier_sem = pltpu.get_barrier_semaphore()
        for nbr in (left, right):
            pl.semaphore_signal(barrier_sem, inc=1, device_id=(nbr,),
                                device_id_type=pl.DeviceIdType.MESH)
        pl.semaphore_wait(barrier_sem, 2)

        def send(src_slot, src_q, dst_slot, sems, idx, dev):
            c = pltpu.make_async_remote_copy(
                src_ref=kv_scr.at[src_slot, src_q:src_q + 2],
                dst_ref=kv_scr.at[dst_slot, src_q:src_q + 2],
                send_sem=sems[0].at[idx], recv_sem=sems[1].at[idx],
                device_id=(dev,), device_id_type=pl.DeviceIdType.MESH)
            c.start()
            return c

        R_SEMS = (send_r, recv_r)
        L_SEMS = (send_l, recv_l)

        r0a = send(0, 0, 1, R_SEMS, 0, right)
        l0b = send(0, 2, 3, L_SEMS, 0, left)
        r0b = send(0, 2, 1, R_SEMS, 1, right)
        l0a = send(0, 0, 3, L_SEMS, 1, left)

        q_scr[...] = (jnp.dot(x_ref[0].astype(BF), wq_ref[...].astype(BF),
                              preferred_element_type=jnp.float32)
                      * SCALE).astype(BF)

        acc_scr[...] = jnp.zeros((S_LOC, H * D), jnp.float32)
        wsum_scr[...] = jnp.zeros((S_LOC, H), jnp.float32)

        R = S_LOC // 2

        def fold(slot, half):
            origin = (my - slot) % N_DEV
            kb = (origin * S_LOC + half * HF
                  + lax.broadcasted_iota(jnp.int32, (1, HF), 1)) // BLK
            for r in range(S_LOC // R):
                rs = slice(r * R, (r + 1) * R)
                qb = (my * S_LOC + r * R
                      + lax.broadcasted_iota(jnp.int32, (R, 1), 0)) // BLK
                mask = (qb == kb) | (kb == 0) | ((qb + kb) % 3 == 0)
                for hd in range(H):
                    sl = slice(hd * D, (hd + 1) * D)
                    s = lax.dot_general(
                        q_scr[rs, sl], kv_scr[slot, 2 * half, :, sl],
                        (((1,), (1,)), ((), ())),
                        preferred_element_type=jnp.float32)
                    w = jnp.where(mask, jnp.exp(s.astype(BF)), BF(0))
                    wsum_scr[rs, hd:hd + 1] = (
                        wsum_scr[rs, hd:hd + 1]
                        + jnp.sum(w, axis=1, keepdims=True,
                                  dtype=jnp.float32))
                    acc_scr[rs, sl] = acc_scr[rs, sl] + lax.dot_general(
                        w, kv_scr[slot, 2 * half + 1, :, sl],
                        (((1,), (0,)), ((), ())),
                        preferred_element_type=jnp.float32)

        fold(0, 0)
        fold(0, 1)

        r0a.wait()
        l0b.wait()
        r1 = send(1, 0, 2, R_SEMS, 2, right)
        l1 = send(3, 2, 2, L_SEMS, 2, left)
        fold(1, 0)
        fold(3, 1)

        r0b.wait()
        l0a.wait()
        fold(1, 1)
        fold(3, 0)

        r1.wait()
        l1.wait()
        fold(2, 0)
        fold(2, 1)

        out_ref[0] = jnp.zeros((S_LOC, H * D), jnp.float32)
        for hd in range(H):
            sl = slice(hd * D, (hd + 1) * D)
            ctx = (acc_scr[:, sl] / wsum_scr[:, hd:hd + 1]).astype(BF)
            out_ref[0] = out_ref[0] + jnp.dot(
                ctx, wo_ref[sl, :].astype(BF),
                preferred_element_type=jnp.float32)

    return pl.pallas_call(
        body,
        out_shape=jax.ShapeDtypeStruct((1, S_LOC, H * D), jnp.float32),
        in_specs=[
            pl.BlockSpec(memory_space=pltpu.VMEM),
            pl.BlockSpec(memory_space=pltpu.VMEM),
            pl.BlockSpec(memory_space=pl.ANY),
            pl.BlockSpec(memory_space=pl.ANY),
            pl.BlockSpec(memory_space=pltpu.VMEM),
        ],
        out_specs=pl.BlockSpec(memory_space=pltpu.VMEM),
        scratch_shapes=[
            pltpu.VMEM((N_DEV, 4, HF, H * D), BF),
            pltpu.VMEM((1, HF // 2, H, D), jnp.float32),
            pltpu.VMEM((S_LOC, H * D), BF),
            pltpu.VMEM((S_LOC, H * D), jnp.float32),
            pltpu.VMEM((S_LOC, H), jnp.float32),
            pltpu.SemaphoreType.DMA,
            pltpu.SemaphoreType.DMA((3,)),
            pltpu.SemaphoreType.DMA((3,)),
            pltpu.SemaphoreType.DMA((3,)),
            pltpu.SemaphoreType.DMA((3,)),
        ],
        compiler_params=pltpu.CompilerParams(
            collective_id=0,
            vmem_limit_bytes=100 * 1024 * 1024,
        ),
    )(x, Wq, K_ext, V_ext, Wo)
